# baseline (speedup 1.0000x reference)
import functools

import jax
import jax.numpy as jnp
from jax import lax
from jax.experimental import pallas as pl
from jax.experimental.pallas import tpu as pltpu
from jax.experimental.pallas import tpu_sc as plsc

_NCLS = 21
_B, _P, _NO = 32, 8732, 8
_PPAD = 9216
_S, _L = 8, 1152
_TH = 0.5
_V0, _V1 = 0.1, 0.2
_NEGPOS = 3
_MAXBITS = 0x7F7FFFFF
_NC = 2


def _mb_kernel(tgt_ref, pri_ref, conf_ref,
               psum_ref, rko_ref, kko_ref,
               ll_ref, pr_ref, np_ref):
    b = pl.program_id(0)

    cx = pri_ref[0]
    cy = pri_ref[1]
    pw = pri_ref[2]
    ph = pri_ref[3]
    px1 = cx - pw * 0.5
    py1 = cy - ph * 0.5
    px2 = cx + pw * 0.5
    py2 = cy + ph * 0.5
    area_p = (px2 - px1) * (py2 - py1)

    pg = (lax.broadcasted_iota(jnp.int32, (_S, _L), 0) * _L
          + lax.broadcasted_iota(jnp.int32, (_S, _L), 1))

    tx1 = [tgt_ref[0, t, 0] for t in range(_NO)]
    ty1 = [tgt_ref[0, t, 1] for t in range(_NO)]
    tx2 = [tgt_ref[0, t, 2] for t in range(_NO)]
    ty2 = [tgt_ref[0, t, 3] for t in range(_NO)]
    tlb = [tgt_ref[0, t, 4] for t in range(_NO)]

    bto = jnp.full((_S, _L), -1.0, dtype=jnp.float32)
    bti = jnp.zeros((_S, _L), dtype=jnp.int32)
    bpi = []
    for t in range(_NO):
        ix1 = jnp.maximum(tx1[t], px1)
        iy1 = jnp.maximum(ty1[t], py1)
        ix2 = jnp.minimum(tx2[t], px2)
        iy2 = jnp.minimum(ty2[t], py2)
        iw = jnp.maximum(ix2 - ix1, 0.0)
        ih = jnp.maximum(iy2 - iy1, 0.0)
        inter = iw * ih
        area_t = (tx2[t] - tx1[t]) * (ty2[t] - ty1[t])
        ov = inter / (area_t + area_p - inter)
        upd = ov > bto
        bti = jnp.where(upd, t, bti)
        bto = jnp.where(upd, ov, bto)
        m = jnp.max(ov)
        bpi.append(jnp.min(jnp.where(ov == m, pg, _PPAD)))

    for t in range(_NO):
        msk = pg == bpi[t]
        bto = jnp.where(msk, 2.0, bto)
        bti = jnp.where(msk, t, bti)

    mx1 = jnp.zeros((_S, _L), jnp.float32)
    my1 = jnp.zeros((_S, _L), jnp.float32)
    mx2 = jnp.zeros((_S, _L), jnp.float32)
    my2 = jnp.zeros((_S, _L), jnp.float32)
    mlb = jnp.zeros((_S, _L), jnp.float32)
    for t in range(_NO):
        sel = bti == t
        mx1 = jnp.where(sel, tx1[t], mx1)
        my1 = jnp.where(sel, ty1[t], my1)
        mx2 = jnp.where(sel, tx2[t], mx2)
        my2 = jnp.where(sel, ty2[t], my2)
        mlb = jnp.where(sel, tlb[t], mlb)

    conf_t = (mlb + 1.0).astype(jnp.int32)
    conf_t = jnp.where(bto < _TH, 0, conf_t)
    pos = conf_t > 0
    posf = pos.astype(jnp.float32)

    g0 = ((mx1 + mx2) * 0.5 - cx) / (_V0 * pw)
    g1 = ((my1 + my2) * 0.5 - cy) / (_V0 * ph)
    g2 = jnp.log((mx2 - mx1) / pw) / _V1
    g3 = jnp.log((my2 - my1) / ph) / _V1
    ll = jnp.float32(0.0)
    for i, g in enumerate((g0, g1, g2, g3)):
        d = conf_ref[0, _NCLS + i] - g
        ad = jnp.abs(d)
        sl1 = jnp.where(ad < 1.0, 0.5 * d * d, ad - 0.5)
        ll = ll + jnp.sum(sl1 * posf)

    x = conf_ref[0, :_NCLS]
    s = jnp.sum(jnp.exp(x), axis=0)
    lse = jnp.log(s)
    x0 = x[0]
    valid = pg < _P
    rk = jnp.where(pos | ~valid, 0.0, lse - x0)

    pos_gather = jnp.zeros((_S, _L), jnp.float32)
    for t in range(_NO):
        ct_t = (tlb[t] + 1.0).astype(jnp.int32)
        x_t = conf_ref[0, ct_t]
        pos_gather = jnp.where(pos & (bti == t), x_t, pos_gather)
    pos_r = jnp.sum(jnp.where(pos, lse, 0.0) - pos_gather)

    num_pos = jnp.sum(conf_t > 0, dtype=jnp.int32)
    k = jnp.minimum(_NEGPOS * num_pos, _P - 1)

    rko_ref[...] = rk[None]
    kko_ref[...] = jnp.full((1, 1, 16), k.astype(jnp.float32), jnp.float32)
    ll_ref[pl.ds(b, 1), :] = jnp.full((1, 1), ll, jnp.float32)
    pr_ref[pl.ds(b, 1), :] = jnp.full((1, 1), pos_r, jnp.float32)
    np_ref[pl.ds(b, 1), :] = jnp.full((1, 1), num_pos, jnp.int32)

    @pl.when(b == _B - 1)
    def _():
        psum_ref[0, 0] = jnp.sum(ll_ref[...])
        psum_ref[0, 1] = jnp.sum(pr_ref[...])
        psum_ref[0, 2] = jnp.sum(np_ref[...].astype(jnp.float32))


def _run_tc(cat, pri, targets, interpret=False):
    return pl.pallas_call(
        _mb_kernel,
        grid=(_B,),
        in_specs=[
            pl.BlockSpec((1, _NO, 5), lambda b: (b, 0, 0),
                         memory_space=pltpu.SMEM),
            pl.BlockSpec((4, _S, _L), lambda b: (0, 0, 0)),
            pl.BlockSpec((1, _NCLS + 4, _S, _L), lambda b: (b, 0, 0, 0)),
        ],
        out_specs=[
            pl.BlockSpec((1, 3), lambda b: (0, 0), memory_space=pltpu.SMEM),
            pl.BlockSpec((1, _S, _L), lambda b: (b, 0, 0)),
            pl.BlockSpec((1, 1, 16), lambda b: (b, 0, 0)),
        ],
        out_shape=[
            jax.ShapeDtypeStruct((1, 3), jnp.float32),
            jax.ShapeDtypeStruct((_B, _S, _L), jnp.float32),
            jax.ShapeDtypeStruct((_B, 1, 16), jnp.float32),
        ],
        scratch_shapes=[
            pltpu.VMEM((_B, 1), jnp.float32),
            pltpu.VMEM((_B, 1), jnp.float32),
            pltpu.VMEM((_B, 1), jnp.int32),
        ],
        interpret=interpret,
    )(targets, pri, cat)


def _run_sc(rk2, kk):
    mesh = plsc.VectorSubcoreMesh(core_axis_name="c", subcore_axis_name="s")

    @functools.partial(
        pl.kernel, mesh=mesh,
        out_type=jax.ShapeDtypeStruct((_B, 48), jnp.float32),
        scratch_types=[
            pltpu.VMEM((_PPAD,), jnp.float32),
            pltpu.VMEM((16,), jnp.float32),
            pltpu.VMEM((48,), jnp.float32),
            pltpu.VMEM((32,), jnp.int32),
        ],
    )
    def sel(rk_hbm, kk_hbm, out_hbm, row_v, kv_v, res_v, tmp_v):
        w = lax.axis_index("s") * _NC + lax.axis_index("c")
        pltpu.sync_copy(rk_hbm.at[w], row_v)
        pltpu.sync_copy(kk_hbm.at[w], kv_v)
        kk_vi = kv_v[...].astype(jnp.int32)

        def lanesum(vec):
            for stride in (1, 2, 4, 8):
                tmp_v[pl.ds(0, 16)] = vec
                tmp_v[pl.ds(16, 16)] = vec
                vec = vec + tmp_v[pl.ds(stride, 16)]
            return vec

        def cnt_ge(mid):
            def step(i, acc):
                v = lax.bitcast_convert_type(row_v[pl.ds(i * 16, 16)],
                                             jnp.int32)
                return acc + jnp.where(v >= mid, 1, 0)
            acc = lax.fori_loop(0, _PPAD // 16, step,
                                jnp.zeros((16,), jnp.int32), unroll=8)
            return lanesum(acc)

        def bs(_, lohi):
            lo, hi = lohi
            mid = lo + lax.shift_right_logical(hi - lo + 1, 1)
            ok = cnt_ge(mid) >= kk_vi
            return (jnp.where(ok, mid, lo), jnp.where(ok, hi, mid - 1))

        lo, _hi = lax.fori_loop(0, 31, bs,
                                (jnp.zeros((16,), jnp.int32),
                                 jnp.full((16,), _MAXBITS, jnp.int32)))

        def fin(i, carry):
            sgt, cgt = carry
            vv = row_v[pl.ds(i * 16, 16)]
            v = lax.bitcast_convert_type(vv, jnp.int32)
            gt = v > lo
            return (sgt + jnp.where(gt, vv, 0.0),
                    cgt + jnp.where(gt, 1, 0))
        sgt, cgt = lax.fori_loop(0, _PPAD // 16, fin,
                                 (jnp.zeros((16,), jnp.float32),
                                  jnp.zeros((16,), jnp.int32)), unroll=8)

        tf = lax.bitcast_convert_type(lo, jnp.float32)
        tf = jnp.where(kk_vi > 0, tf, 0.0)
        res_v[pl.ds(0, 16)] = sgt
        res_v[pl.ds(16, 16)] = cgt.astype(jnp.float32)
        res_v[pl.ds(32, 16)] = tf
        pltpu.sync_copy(res_v, out_hbm.at[w])

    return sel(rk2, kk)


def _prep(mbd1_loc_data, mbd1_conf_data, priors):
    npad = _PPAD - _P
    pad_rows = jnp.tile(
        jnp.array([[-100.0, -100.0, 1.0, 1.0]], jnp.float32), (npad, 1))
    pri = jnp.concatenate([priors, pad_rows], axis=0)
    pri = pri.T.reshape(4, _S, _L)
    cat = jnp.concatenate([mbd1_conf_data, mbd1_loc_data], axis=2)
    cat = jnp.pad(cat, ((0, 0), (0, npad), (0, 0)))
    cat = jnp.transpose(cat, (0, 2, 1)).reshape(_B, _NCLS + 4, _S, _L)
    return cat, pri


def kernel(mbd1_loc_data, mbd1_conf_data, mbd2_loc_data, mbd2_conf_data,
           priors, targets):
    del mbd2_loc_data, mbd2_conf_data
    cat, pri = _prep(mbd1_loc_data, mbd1_conf_data, priors)
    psum, rk, kk = _run_tc(cat, pri, targets)
    sc = _run_sc(rk.reshape(_B, _PPAD), kk.reshape(_B, 16))
    k_row = kk.reshape(_B, 16)[:, 0]
    s_gt = jnp.sum(sc[:, 0:16], axis=1)
    cnt_gt = jnp.sum(sc[:, 16:32], axis=1)
    tf = sc[:, 32]
    extra = s_gt + (k_row - cnt_gt) * tf
    n_total = psum[0, 2]
    loss_l = psum[0, 0] / n_total
    loss_c = (psum[0, 1] + jnp.sum(extra)) / n_total
    return loss_l, loss_c


# speedup vs baseline: 1.0605x; 1.0605x over previous
import functools

import jax
import jax.numpy as jnp
from jax import lax
from jax.experimental import pallas as pl
from jax.experimental.pallas import tpu as pltpu
from jax.experimental.pallas import tpu_sc as plsc

_NCLS = 21
_B, _P, _NO = 32, 8732, 8
_PPAD = 9216
_S, _L = 8, 1152
_TH = 0.5
_V0, _V1 = 0.1, 0.2
_NEGPOS = 3
_MAXBITS = 0x7F7FFFFF
_NC = 2
_DB = 2


def _one_batch(db, tgt_ref, loc_ref, conf_ref,
               cx, cy, pw, ph, px1, py1, px2, py2, area_p, pg):
    tx1 = [tgt_ref[db, t, 0] for t in range(_NO)]
    ty1 = [tgt_ref[db, t, 1] for t in range(_NO)]
    tx2 = [tgt_ref[db, t, 2] for t in range(_NO)]
    ty2 = [tgt_ref[db, t, 3] for t in range(_NO)]
    tlb = [tgt_ref[db, t, 4] for t in range(_NO)]

    bto = jnp.full((_S, _L), -1.0, dtype=jnp.float32)
    bti = jnp.zeros((_S, _L), dtype=jnp.int32)
    bpi = []
    for t in range(_NO):
        ix1 = jnp.maximum(tx1[t], px1)
        iy1 = jnp.maximum(ty1[t], py1)
        ix2 = jnp.minimum(tx2[t], px2)
        iy2 = jnp.minimum(ty2[t], py2)
        iw = jnp.maximum(ix2 - ix1, 0.0)
        ih = jnp.maximum(iy2 - iy1, 0.0)
        inter = iw * ih
        area_t = (tx2[t] - tx1[t]) * (ty2[t] - ty1[t])
        ov = inter / (area_t + area_p - inter)
        upd = ov > bto
        bti = jnp.where(upd, t, bti)
        bto = jnp.where(upd, ov, bto)
        m = jnp.max(ov)
        bpi.append(jnp.min(jnp.where(ov == m, pg, _PPAD)))

    for t in range(_NO):
        msk = pg == bpi[t]
        bto = jnp.where(msk, 2.0, bto)
        bti = jnp.where(msk, t, bti)

    mx1 = jnp.zeros((_S, _L), jnp.float32)
    my1 = jnp.zeros((_S, _L), jnp.float32)
    mx2 = jnp.zeros((_S, _L), jnp.float32)
    my2 = jnp.zeros((_S, _L), jnp.float32)
    mlb = jnp.zeros((_S, _L), jnp.float32)
    for t in range(_NO):
        sel = bti == t
        mx1 = jnp.where(sel, tx1[t], mx1)
        my1 = jnp.where(sel, ty1[t], my1)
        mx2 = jnp.where(sel, tx2[t], mx2)
        my2 = jnp.where(sel, ty2[t], my2)
        mlb = jnp.where(sel, tlb[t], mlb)

    conf_t = (mlb + 1.0).astype(jnp.int32)
    conf_t = jnp.where(bto < _TH, 0, conf_t)
    pos = conf_t > 0
    posf = pos.astype(jnp.float32)

    g0 = ((mx1 + mx2) * 0.5 - cx) / (_V0 * pw)
    g1 = ((my1 + my2) * 0.5 - cy) / (_V0 * ph)
    g2 = jnp.log((mx2 - mx1) / pw) / _V1
    g3 = jnp.log((my2 - my1) / ph) / _V1
    ll = jnp.float32(0.0)
    for i, g in enumerate((g0, g1, g2, g3)):
        d = loc_ref[db, i] - g
        ad = jnp.abs(d)
        sl1 = jnp.where(ad < 1.0, 0.5 * d * d, ad - 0.5)
        ll = ll + jnp.sum(sl1 * posf)

    x = conf_ref[db]
    s = jnp.sum(jnp.exp(x), axis=0)
    lse = jnp.log(s)
    x0 = x[0]
    valid = pg < _P
    rk = jnp.where(pos | ~valid, 0.0, lse - x0)

    pos_gather = jnp.zeros((_S, _L), jnp.float32)
    for t in range(_NO):
        ct_t = (tlb[t] + 1.0).astype(jnp.int32)
        x_t = conf_ref[db, ct_t]
        pos_gather = jnp.where(pos & (bti == t), x_t, pos_gather)
    pos_r = jnp.sum(jnp.where(pos, lse, 0.0) - pos_gather)

    num_pos = jnp.sum(conf_t > 0, dtype=jnp.int32)
    k = jnp.minimum(_NEGPOS * num_pos, _P - 1)
    return rk, k, ll, pos_r, num_pos


def _mb_kernel(tgt_ref, pri_ref, loc_ref, conf_ref,
               psum_ref, rko_ref, kko_ref,
               ll_ref, pr_ref, np_ref):
    g = pl.program_id(0)

    cx = pri_ref[0]
    cy = pri_ref[1]
    pw = pri_ref[2]
    ph = pri_ref[3]
    px1 = cx - pw * 0.5
    py1 = cy - ph * 0.5
    px2 = cx + pw * 0.5
    py2 = cy + ph * 0.5
    area_p = (px2 - px1) * (py2 - py1)

    pg = (lax.broadcasted_iota(jnp.int32, (_S, _L), 0) * _L
          + lax.broadcasted_iota(jnp.int32, (_S, _L), 1))

    for db in range(_DB):
        rk, k, ll, pos_r, num_pos = _one_batch(
            db, tgt_ref, loc_ref, conf_ref,
            cx, cy, pw, ph, px1, py1, px2, py2, area_p, pg)
        b = g * _DB + db
        rko_ref[db] = rk
        kko_ref[db] = jnp.full((1, 16), k.astype(jnp.float32), jnp.float32)
        ll_ref[pl.ds(b, 1), :] = jnp.full((1, 1), ll, jnp.float32)
        pr_ref[pl.ds(b, 1), :] = jnp.full((1, 1), pos_r, jnp.float32)
        np_ref[pl.ds(b, 1), :] = jnp.full((1, 1), num_pos, jnp.int32)

    @pl.when(g == _B // _DB - 1)
    def _():
        psum_ref[0, 0] = jnp.sum(ll_ref[...])
        psum_ref[0, 1] = jnp.sum(pr_ref[...])
        psum_ref[0, 2] = jnp.sum(np_ref[...].astype(jnp.float32))


def _run_tc(loc, conf, pri, targets, interpret=False):
    return pl.pallas_call(
        _mb_kernel,
        grid=(_B // _DB,),
        in_specs=[
            pl.BlockSpec((_DB, _NO, 5), lambda b: (b, 0, 0),
                         memory_space=pltpu.SMEM),
            pl.BlockSpec((4, _S, _L), lambda b: (0, 0, 0)),
            pl.BlockSpec((_DB, 4, _S, _L), lambda b: (b, 0, 0, 0)),
            pl.BlockSpec((_DB, _NCLS, _S, _L), lambda b: (b, 0, 0, 0)),
        ],
        out_specs=[
            pl.BlockSpec((1, 3), lambda b: (0, 0), memory_space=pltpu.SMEM),
            pl.BlockSpec((_DB, _S, _L), lambda b: (b, 0, 0)),
            pl.BlockSpec((_DB, 1, 16), lambda b: (b, 0, 0)),
        ],
        out_shape=[
            jax.ShapeDtypeStruct((1, 3), jnp.float32),
            jax.ShapeDtypeStruct((_B, _S, _L), jnp.float32),
            jax.ShapeDtypeStruct((_B, 1, 16), jnp.float32),
        ],
        scratch_shapes=[
            pltpu.VMEM((_B, 1), jnp.float32),
            pltpu.VMEM((_B, 1), jnp.float32),
            pltpu.VMEM((_B, 1), jnp.int32),
        ],
        interpret=interpret,
    )(targets, pri, loc, conf)


def _run_sc(rk2, kk):
    mesh = plsc.VectorSubcoreMesh(core_axis_name="c", subcore_axis_name="s")

    @functools.partial(
        pl.kernel, mesh=mesh,
        out_type=jax.ShapeDtypeStruct((_B, 48), jnp.float32),
        scratch_types=[
            pltpu.VMEM((_PPAD,), jnp.float32),
            pltpu.VMEM((16,), jnp.float32),
            pltpu.VMEM((48,), jnp.float32),
            pltpu.VMEM((32,), jnp.int32),
        ],
    )
    def sel(rk_hbm, kk_hbm, out_hbm, row_v, kv_v, res_v, tmp_v):
        w = lax.axis_index("s") * _NC + lax.axis_index("c")
        pltpu.sync_copy(rk_hbm.at[w], row_v)
        pltpu.sync_copy(kk_hbm.at[w], kv_v)
        kk_vi = kv_v[...].astype(jnp.int32)

        def lanesum(vec):
            for stride in (1, 2, 4, 8):
                tmp_v[pl.ds(0, 16)] = vec
                tmp_v[pl.ds(16, 16)] = vec
                vec = vec + tmp_v[pl.ds(stride, 16)]
            return vec

        def cnt_ge(mid):
            def step(i, acc):
                v = lax.bitcast_convert_type(row_v[pl.ds(i * 16, 16)],
                                             jnp.int32)
                return acc + jnp.where(v >= mid, 1, 0)
            acc = lax.fori_loop(0, _PPAD // 16, step,
                                jnp.zeros((16,), jnp.int32), unroll=8)
            return lanesum(acc)

        def bs(_, lohi):
            lo, hi = lohi
            mid = lo + lax.shift_right_logical(hi - lo + 1, 1)
            ok = cnt_ge(mid) >= kk_vi
            return (jnp.where(ok, mid, lo), jnp.where(ok, hi, mid - 1))

        lo, _hi = lax.fori_loop(0, 31, bs,
                                (jnp.zeros((16,), jnp.int32),
                                 jnp.full((16,), _MAXBITS, jnp.int32)))

        def fin(i, carry):
            sgt, cgt = carry
            vv = row_v[pl.ds(i * 16, 16)]
            v = lax.bitcast_convert_type(vv, jnp.int32)
            gt = v > lo
            return (sgt + jnp.where(gt, vv, 0.0),
                    cgt + jnp.where(gt, 1, 0))
        sgt, cgt = lax.fori_loop(0, _PPAD // 16, fin,
                                 (jnp.zeros((16,), jnp.float32),
                                  jnp.zeros((16,), jnp.int32)), unroll=8)

        tf = lax.bitcast_convert_type(lo, jnp.float32)
        tf = jnp.where(kk_vi > 0, tf, 0.0)
        res_v[pl.ds(0, 16)] = sgt
        res_v[pl.ds(16, 16)] = cgt.astype(jnp.float32)
        res_v[pl.ds(32, 16)] = tf
        pltpu.sync_copy(res_v, out_hbm.at[w])

    return sel(rk2, kk)


def _prep(mbd1_loc_data, mbd1_conf_data, priors):
    npad = _PPAD - _P
    pad_rows = jnp.tile(
        jnp.array([[-100.0, -100.0, 1.0, 1.0]], jnp.float32), (npad, 1))
    pri = jnp.concatenate([priors, pad_rows], axis=0)
    pri = pri.T.reshape(4, _S, _L)
    loc = jnp.pad(mbd1_loc_data, ((0, 0), (0, npad), (0, 0)))
    loc = jnp.transpose(loc, (0, 2, 1)).reshape(_B, 4, _S, _L)
    conf = jnp.pad(mbd1_conf_data, ((0, 0), (0, npad), (0, 0)))
    conf = jnp.transpose(conf, (0, 2, 1)).reshape(_B, _NCLS, _S, _L)
    return loc, conf, pri


def kernel(mbd1_loc_data, mbd1_conf_data, mbd2_loc_data, mbd2_conf_data,
           priors, targets):
    del mbd2_loc_data, mbd2_conf_data
    loc, conf, pri = _prep(mbd1_loc_data, mbd1_conf_data, priors)
    psum, rk, kk = _run_tc(loc, conf, pri, targets)
    sc = _run_sc(rk.reshape(_B, _PPAD), kk.reshape(_B, 16))
    k_row = kk.reshape(_B, 16)[:, 0]
    s_gt = jnp.sum(sc[:, 0:16], axis=1)
    cnt_gt = jnp.sum(sc[:, 16:32], axis=1)
    tf = sc[:, 32]
    extra = s_gt + (k_row - cnt_gt) * tf
    n_total = psum[0, 2]
    loss_l = psum[0, 0] / n_total
    loss_c = (psum[0, 1] + jnp.sum(extra)) / n_total
    return loss_l, loss_c


# speedup vs baseline: 1.0703x; 1.0092x over previous
import functools

import jax
import jax.numpy as jnp
from jax import lax
from jax.experimental import pallas as pl
from jax.experimental.pallas import tpu as pltpu
from jax.experimental.pallas import tpu_sc as plsc

_NCLS = 21
_B, _P, _NO = 32, 8732, 8
_PPAD = 9216
_S, _L = 8, 1152
_TH = 0.5
_V0, _V1 = 0.1, 0.2
_NEGPOS = 3
_MAXBITS = 0x7F7FFFFF
_NC = 2
_DB = 4


def _one_batch(db, tgt_ref, loc_ref, conf_ref,
               cx, cy, pw, ph, px1, py1, px2, py2, area_p, pg):
    tx1 = [tgt_ref[db, t, 0] for t in range(_NO)]
    ty1 = [tgt_ref[db, t, 1] for t in range(_NO)]
    tx2 = [tgt_ref[db, t, 2] for t in range(_NO)]
    ty2 = [tgt_ref[db, t, 3] for t in range(_NO)]
    tlb = [tgt_ref[db, t, 4] for t in range(_NO)]

    bto = jnp.full((_S, _L), -1.0, dtype=jnp.float32)
    bti = jnp.zeros((_S, _L), dtype=jnp.int32)
    bpi = []
    for t in range(_NO):
        ix1 = jnp.maximum(tx1[t], px1)
        iy1 = jnp.maximum(ty1[t], py1)
        ix2 = jnp.minimum(tx2[t], px2)
        iy2 = jnp.minimum(ty2[t], py2)
        iw = jnp.maximum(ix2 - ix1, 0.0)
        ih = jnp.maximum(iy2 - iy1, 0.0)
        inter = iw * ih
        area_t = (tx2[t] - tx1[t]) * (ty2[t] - ty1[t])
        ov = inter / (area_t + area_p - inter)
        upd = ov > bto
        bti = jnp.where(upd, t, bti)
        bto = jnp.where(upd, ov, bto)
        m = jnp.max(ov)
        bpi.append(jnp.min(jnp.where(ov == m, pg, _PPAD)))

    for t in range(_NO):
        msk = pg == bpi[t]
        bto = jnp.where(msk, 2.0, bto)
        bti = jnp.where(msk, t, bti)

    mx1 = jnp.zeros((_S, _L), jnp.float32)
    my1 = jnp.zeros((_S, _L), jnp.float32)
    mx2 = jnp.zeros((_S, _L), jnp.float32)
    my2 = jnp.zeros((_S, _L), jnp.float32)
    mlb = jnp.zeros((_S, _L), jnp.float32)
    for t in range(_NO):
        sel = bti == t
        mx1 = jnp.where(sel, tx1[t], mx1)
        my1 = jnp.where(sel, ty1[t], my1)
        mx2 = jnp.where(sel, tx2[t], mx2)
        my2 = jnp.where(sel, ty2[t], my2)
        mlb = jnp.where(sel, tlb[t], mlb)

    conf_t = (mlb + 1.0).astype(jnp.int32)
    conf_t = jnp.where(bto < _TH, 0, conf_t)
    pos = conf_t > 0
    posf = pos.astype(jnp.float32)

    g0 = ((mx1 + mx2) * 0.5 - cx) / (_V0 * pw)
    g1 = ((my1 + my2) * 0.5 - cy) / (_V0 * ph)
    g2 = jnp.log((mx2 - mx1) / pw) / _V1
    g3 = jnp.log((my2 - my1) / ph) / _V1
    ll = jnp.float32(0.0)
    for i, g in enumerate((g0, g1, g2, g3)):
        d = loc_ref[db, i] - g
        ad = jnp.abs(d)
        sl1 = jnp.where(ad < 1.0, 0.5 * d * d, ad - 0.5)
        ll = ll + jnp.sum(sl1 * posf)

    x = conf_ref[db]
    s = jnp.sum(jnp.exp(x), axis=0)
    lse = jnp.log(s)
    x0 = x[0]
    valid = pg < _P
    rk = jnp.where(pos | ~valid, 0.0, lse - x0)

    pos_gather = jnp.zeros((_S, _L), jnp.float32)
    for t in range(_NO):
        ct_t = (tlb[t] + 1.0).astype(jnp.int32)
        x_t = conf_ref[db, ct_t]
        pos_gather = jnp.where(pos & (bti == t), x_t, pos_gather)
    pos_r = jnp.sum(jnp.where(pos, lse, 0.0) - pos_gather)

    num_pos = jnp.sum(conf_t > 0, dtype=jnp.int32)
    k = jnp.minimum(_NEGPOS * num_pos, _P - 1)
    return rk, k, ll, pos_r, num_pos


def _mb_kernel(tgt_ref, pri_ref, loc_ref, conf_ref,
               psum_ref, rko_ref, kko_ref,
               ll_ref, pr_ref, np_ref):
    g = pl.program_id(0)

    cx = pri_ref[0]
    cy = pri_ref[1]
    pw = pri_ref[2]
    ph = pri_ref[3]
    px1 = cx - pw * 0.5
    py1 = cy - ph * 0.5
    px2 = cx + pw * 0.5
    py2 = cy + ph * 0.5
    area_p = (px2 - px1) * (py2 - py1)

    pg = (lax.broadcasted_iota(jnp.int32, (_S, _L), 0) * _L
          + lax.broadcasted_iota(jnp.int32, (_S, _L), 1))

    for db in range(_DB):
        rk, k, ll, pos_r, num_pos = _one_batch(
            db, tgt_ref, loc_ref, conf_ref,
            cx, cy, pw, ph, px1, py1, px2, py2, area_p, pg)
        b = g * _DB + db
        rko_ref[db] = rk
        kko_ref[db] = jnp.full((1, 16), k.astype(jnp.float32), jnp.float32)
        ll_ref[pl.ds(b, 1), :] = jnp.full((1, 1), ll, jnp.float32)
        pr_ref[pl.ds(b, 1), :] = jnp.full((1, 1), pos_r, jnp.float32)
        np_ref[pl.ds(b, 1), :] = jnp.full((1, 1), num_pos, jnp.int32)

    @pl.when(g == _B // _DB - 1)
    def _():
        psum_ref[0, 0] = jnp.sum(ll_ref[...])
        psum_ref[0, 1] = jnp.sum(pr_ref[...])
        psum_ref[0, 2] = jnp.sum(np_ref[...].astype(jnp.float32))


def _run_tc(loc, conf, pri, targets, interpret=False):
    return pl.pallas_call(
        _mb_kernel,
        grid=(_B // _DB,),
        in_specs=[
            pl.BlockSpec((_DB, _NO, 5), lambda b: (b, 0, 0),
                         memory_space=pltpu.SMEM),
            pl.BlockSpec((4, _S, _L), lambda b: (0, 0, 0)),
            pl.BlockSpec((_DB, 4, _S, _L), lambda b: (b, 0, 0, 0)),
            pl.BlockSpec((_DB, _NCLS, _S, _L), lambda b: (b, 0, 0, 0)),
        ],
        out_specs=[
            pl.BlockSpec((1, 3), lambda b: (0, 0), memory_space=pltpu.SMEM),
            pl.BlockSpec((_DB, _S, _L), lambda b: (b, 0, 0)),
            pl.BlockSpec((_DB, 1, 16), lambda b: (b, 0, 0)),
        ],
        out_shape=[
            jax.ShapeDtypeStruct((1, 3), jnp.float32),
            jax.ShapeDtypeStruct((_B, _S, _L), jnp.float32),
            jax.ShapeDtypeStruct((_B, 1, 16), jnp.float32),
        ],
        scratch_shapes=[
            pltpu.VMEM((_B, 1), jnp.float32),
            pltpu.VMEM((_B, 1), jnp.float32),
            pltpu.VMEM((_B, 1), jnp.int32),
        ],
        interpret=interpret,
    )(targets, pri, loc, conf)


def _run_sc(rk2, kk):
    mesh = plsc.VectorSubcoreMesh(core_axis_name="c", subcore_axis_name="s")

    @functools.partial(
        pl.kernel, mesh=mesh,
        out_type=jax.ShapeDtypeStruct((_B, 48), jnp.float32),
        scratch_types=[
            pltpu.VMEM((_PPAD,), jnp.float32),
            pltpu.VMEM((16,), jnp.float32),
            pltpu.VMEM((48,), jnp.float32),
            pltpu.VMEM((32,), jnp.int32),
        ],
    )
    def sel(rk_hbm, kk_hbm, out_hbm, row_v, kv_v, res_v, tmp_v):
        w = lax.axis_index("s") * _NC + lax.axis_index("c")
        pltpu.sync_copy(rk_hbm.at[w], row_v)
        pltpu.sync_copy(kk_hbm.at[w], kv_v)
        kk_vi = kv_v[...].astype(jnp.int32)

        def lanesum(vec):
            for stride in (1, 2, 4, 8):
                tmp_v[pl.ds(0, 16)] = vec
                tmp_v[pl.ds(16, 16)] = vec
                vec = vec + tmp_v[pl.ds(stride, 16)]
            return vec

        def cnt_ge(mid):
            def step(i, acc):
                v = lax.bitcast_convert_type(row_v[pl.ds(i * 16, 16)],
                                             jnp.int32)
                return acc + jnp.where(v >= mid, 1, 0)
            acc = lax.fori_loop(0, _PPAD // 16, step,
                                jnp.zeros((16,), jnp.int32), unroll=8)
            return lanesum(acc)

        def bs(_, lohi):
            lo, hi = lohi
            mid = lo + lax.shift_right_logical(hi - lo + 1, 1)
            ok = cnt_ge(mid) >= kk_vi
            return (jnp.where(ok, mid, lo), jnp.where(ok, hi, mid - 1))

        lo, _hi = lax.fori_loop(0, 31, bs,
                                (jnp.zeros((16,), jnp.int32),
                                 jnp.full((16,), _MAXBITS, jnp.int32)))

        def fin(i, carry):
            sgt, cgt = carry
            vv = row_v[pl.ds(i * 16, 16)]
            v = lax.bitcast_convert_type(vv, jnp.int32)
            gt = v > lo
            return (sgt + jnp.where(gt, vv, 0.0),
                    cgt + jnp.where(gt, 1, 0))
        sgt, cgt = lax.fori_loop(0, _PPAD // 16, fin,
                                 (jnp.zeros((16,), jnp.float32),
                                  jnp.zeros((16,), jnp.int32)), unroll=8)

        tf = lax.bitcast_convert_type(lo, jnp.float32)
        tf = jnp.where(kk_vi > 0, tf, 0.0)
        res_v[pl.ds(0, 16)] = sgt
        res_v[pl.ds(16, 16)] = cgt.astype(jnp.float32)
        res_v[pl.ds(32, 16)] = tf
        pltpu.sync_copy(res_v, out_hbm.at[w])

    return sel(rk2, kk)


def _prep(mbd1_loc_data, mbd1_conf_data, priors):
    npad = _PPAD - _P
    pad_rows = jnp.tile(
        jnp.array([[-100.0, -100.0, 1.0, 1.0]], jnp.float32), (npad, 1))
    pri = jnp.concatenate([priors, pad_rows], axis=0)
    pri = pri.T.reshape(4, _S, _L)
    loc = jnp.pad(mbd1_loc_data, ((0, 0), (0, npad), (0, 0)))
    loc = jnp.transpose(loc, (0, 2, 1)).reshape(_B, 4, _S, _L)
    conf = jnp.pad(mbd1_conf_data, ((0, 0), (0, npad), (0, 0)))
    conf = jnp.transpose(conf, (0, 2, 1)).reshape(_B, _NCLS, _S, _L)
    return loc, conf, pri


def kernel(mbd1_loc_data, mbd1_conf_data, mbd2_loc_data, mbd2_conf_data,
           priors, targets):
    del mbd2_loc_data, mbd2_conf_data
    loc, conf, pri = _prep(mbd1_loc_data, mbd1_conf_data, priors)
    psum, rk, kk = _run_tc(loc, conf, pri, targets)
    sc = _run_sc(rk.reshape(_B, _PPAD), kk.reshape(_B, 16))
    k_row = kk.reshape(_B, 16)[:, 0]
    s_gt = jnp.sum(sc[:, 0:16], axis=1)
    cnt_gt = jnp.sum(sc[:, 16:32], axis=1)
    tf = sc[:, 32]
    extra = s_gt + (k_row - cnt_gt) * tf
    n_total = psum[0, 2]
    loss_l = psum[0, 0] / n_total
    loss_c = (psum[0, 1] + jnp.sum(extra)) / n_total
    return loss_l, loss_c
